# Initial kernel scaffold; baseline (speedup 1.0000x reference)
import jax
import jax.numpy as jnp
from jax.experimental import pallas as pl


def kernel(X, A, W0, b0, W1, b1, W2, b2):
    raise NotImplementedError("write your pallas kernel here")


# speedup vs baseline: 11.2237x; 11.2237x over previous
import functools

import jax
import jax.numpy as jnp
from jax import lax
from jax.experimental import pallas as pl
from jax.experimental.pallas import tpu as pltpu
from jax.experimental.pallas import tpu_sc as plsc

N = 10000
D = 128
NCLS = 40
E = 320000

NP = 10240
NCORE = 2
NSUB = 16
NW = NCORE * NSUB
K = 128
NCHUNK_W = 79
EP = NW * NCHUNK_W * K
ROWS_PER_SUB = NP // NSUB

BN = 1280
GRID = NP // BN


def _mesh():
    return plsc.VectorSubcoreMesh(core_axis_name="c", subcore_axis_name="s",
                                  num_cores=NCORE, num_subcores=NSUB)


def _hist_body(dst3, zeros_n, ones_k, out, dst_v, ones_v, hist_sp):
    c = lax.axis_index("c")
    s = lax.axis_index("s")
    wid = c * NSUB + s
    off = s * ROWS_PER_SUB
    pltpu.sync_copy(zeros_n.at[pl.ds(off, ROWS_PER_SUB)],
                    hist_sp.at[pl.ds(off, ROWS_PER_SUB)])
    pltpu.sync_copy(ones_k, ones_v)
    pltpu.sync_copy(dst3.at[wid], dst_v)
    plsc.subcore_barrier()

    def body(j, carry):
        pltpu.sync_copy(ones_v, hist_sp.at[dst_v.at[j]], add=True)
        return carry

    lax.fori_loop(0, NCHUNK_W, body, 0)
    plsc.subcore_barrier()
    pltpu.sync_copy(hist_sp.at[pl.ds(off, ROWS_PER_SUB)],
                    out.at[pl.ds(c * NP + off, ROWS_PER_SUB)])


_hist = functools.partial(
    pl.kernel,
    out_type=jax.ShapeDtypeStruct((NCORE * NP,), jnp.float32),
    mesh=_mesh(),
    scratch_types=[
        pltpu.VMEM((NCHUNK_W, K), jnp.int32),
        pltpu.VMEM((K,), jnp.float32),
        pltpu.VMEM_SHARED((NP,), jnp.float32),
    ],
)(_hist_body)


def _agg_body(table, src3, dst3, zeros_t, out, src_v, dst_v, buf, acc, sem):
    c = lax.axis_index("c")
    s = lax.axis_index("s")
    wid = c * NSUB + s
    off = s * ROWS_PER_SUB
    pltpu.sync_copy(zeros_t.at[pl.ds(off, ROWS_PER_SUB)],
                    acc.at[pl.ds(off, ROWS_PER_SUB)])
    pltpu.sync_copy(src3.at[wid], src_v)
    pltpu.sync_copy(dst3.at[wid], dst_v)
    plsc.subcore_barrier()

    def body(j, carry):
        pltpu.async_copy(table.at[src_v.at[j]], buf, sem).wait()
        pltpu.sync_copy(buf, acc.at[dst_v.at[j]], add=True)
        return carry

    lax.fori_loop(0, NCHUNK_W, body, 0)
    plsc.subcore_barrier()
    pltpu.sync_copy(acc.at[pl.ds(off, ROWS_PER_SUB)],
                    out.at[c, pl.ds(off, ROWS_PER_SUB)])


_agg = functools.partial(
    pl.kernel,
    out_type=jax.ShapeDtypeStruct((NCORE, NP, D), jnp.float32),
    mesh=_mesh(),
    scratch_types=[
        pltpu.VMEM((NCHUNK_W, K), jnp.int32),
        pltpu.VMEM((NCHUNK_W, K), jnp.int32),
        pltpu.VMEM((K, D), jnp.float32),
        pltpu.VMEM_SHARED((NP, D), jnp.float32),
        pltpu.SemaphoreType.DMA,
    ],
)(_agg_body)


def _dinv(h_ref):
    return lax.rsqrt(1.0 + h_ref[:, 0:1] + h_ref[:, 1:2])


def _tc1_body(x_ref, w_ref, h_ref, o_ref):
    o_ref[...] = jnp.dot(x_ref[...], w_ref[...],
                         preferred_element_type=jnp.float32) * _dinv(h_ref)


def _tc2_body(acc_ref, hp_ref, h_ref, b_ref, w_ref, o_ref):
    dinv = _dinv(h_ref)
    pre = (acc_ref[0] + acc_ref[1] + hp_ref[...]) * dinv + b_ref[...]
    h = jnp.maximum(pre, 0.0)
    o_ref[...] = jnp.dot(h, w_ref[...],
                         preferred_element_type=jnp.float32) * dinv


def _tc3_body(acc_ref, hp_ref, h_ref, b_ref, o_ref):
    dinv = _dinv(h_ref)
    pre = (acc_ref[0] + acc_ref[1] + hp_ref[...]) * dinv + b_ref[...]
    o_ref[...] = jnp.maximum(pre, 0.0) * dinv


def _tc4_body(acc_ref, hp_ref, h_ref, w_ref, b_ref, o_ref):
    pre = (acc_ref[0] + acc_ref[1] + hp_ref[...]) * _dinv(h_ref)
    o_ref[...] = jnp.dot(pre, w_ref[...],
                         preferred_element_type=jnp.float32) + b_ref[...]


def _row_spec(width):
    return pl.BlockSpec((BN, width), lambda i: (i, 0))


_acc_spec = pl.BlockSpec((NCORE, BN, D), lambda i: (0, i, 0))
_w_spec = pl.BlockSpec((D, D), lambda i: (0, 0))
_b_spec = pl.BlockSpec((1, D), lambda i: (0, 0))
_out_struct = jax.ShapeDtypeStruct((NP, D), jnp.float32)

_tc1 = pl.pallas_call(
    _tc1_body, grid=(GRID,),
    in_specs=[_row_spec(D), _w_spec, _row_spec(2)],
    out_specs=_row_spec(D), out_shape=_out_struct)

_tc2 = pl.pallas_call(
    _tc2_body, grid=(GRID,),
    in_specs=[_acc_spec, _row_spec(D), _row_spec(2), _b_spec, _w_spec],
    out_specs=_row_spec(D), out_shape=_out_struct)

_tc3 = pl.pallas_call(
    _tc3_body, grid=(GRID,),
    in_specs=[_acc_spec, _row_spec(D), _row_spec(2), _b_spec],
    out_specs=_row_spec(D), out_shape=_out_struct)

_tc4 = pl.pallas_call(
    _tc4_body, grid=(GRID,),
    in_specs=[_acc_spec, _row_spec(D), _row_spec(2), _w_spec, _b_spec],
    out_specs=_row_spec(D), out_shape=_out_struct)


def kernel(X, A, W0, b0, W1, b1, W2, b2):
    src = A[0].astype(jnp.int32)
    dst = A[1].astype(jnp.int32)
    pad = EP - E
    srcp = jnp.concatenate([src, jnp.zeros((pad,), jnp.int32)])
    dstp = jnp.concatenate([dst, jnp.full((pad,), N, jnp.int32)])
    src3 = srcp.reshape(NW, NCHUNK_W, K)
    dst3 = dstp.reshape(NW, NCHUNK_W, K)

    Xp = jnp.pad(X, ((0, NP - N), (0, 0)))
    zeros_t = jnp.zeros((NP, D), jnp.float32)
    zeros_n = jnp.zeros((NP,), jnp.float32)
    ones_k = jnp.ones((K,), jnp.float32)

    hist = _hist(dst3, zeros_n, ones_k).reshape(NCORE, NP)
    histT = hist.T

    h0 = _tc1(Xp, W0, histT)
    a0 = _agg(h0, src3, dst3, zeros_t)
    h1 = _tc2(a0, h0, histT, b0.reshape(1, D), W1)
    a1 = _agg(h1, src3, dst3, zeros_t)
    h2 = _tc3(a1, h1, histT, b1.reshape(1, D))
    a2 = _agg(h2, src3, dst3, zeros_t)
    W2p = jnp.pad(W2, ((0, 0), (0, D - NCLS)))
    b2p = jnp.pad(b2, (0, D - NCLS)).reshape(1, D)
    outp = _tc4(a2, h2, histT, W2p, b2p)
    return outp[:N, :NCLS]
